# baseline (speedup 1.0000x reference)
import jax
import jax.numpy as jnp
from jax import lax
from jax.experimental import pallas as pl
from jax.experimental.pallas import tpu as pltpu
from jax.experimental.pallas import tpu_sc as plsc

N_USERS = 25000
N_ITEMS = 25000
N_NODES = N_USERS + N_ITEMS
N_EDGES = 800000
HALF = 32
BATCH = 16384

NC = 2
NS = 16
CHUNK = 128
GCH = 2
E_SLAB = CHUNK * GCH
N_EDGES_PAD = 835584
N_SLABS = N_EDGES_PAD // E_SLAB
SLABS_SUB = N_SLABS // NS
M_LOOPS = SLABS_SUB // 6
PIECE = 250
N_PIECES = N_NODES // PIECE
P_SUB = BATCH // NS
PCHUNK = 64
P_LOOPS = P_SUB // PCHUNK


def _sc_body(init_ref, packed_ref, users_ref, items_ref,
             gamma_ref, l1_ref, l2_ref, l3_ref,
             acc, pb0, pb1, pb2, pb3, pb4, pb5, gb0, gb1, gb2, uv, iv,
             gammav, gs0, gs1, gs2, ss0, ss1, ss2, isem):
  c = lax.axis_index("c")
  s = lax.axis_index("s")
  pbufs = (pb0, pb1, pb2, pb3, pb4, pb5)
  gbufs = (gb0, gb1, gb2)
  gsems = (gs0, gs1, gs2)
  ssems = (ss0, ss1, ss2)

  def propagate(src_tbl, dst_tbl):
    @pl.loop(0, PIECE)
    def _(r):
      gb2[r, pl.ds(0, 16)] = jnp.zeros((16,), jnp.float32)
      gb2[r, pl.ds(16, 16)] = jnp.zeros((16,), jnp.float32)

    @pl.loop(s, N_PIECES, step=NS)
    def _(j):
      pltpu.sync_copy(gb2.at[pl.ds(0, PIECE)],
                      acc.at[pl.ds(j * PIECE, PIECE)])
    plsc.subcore_barrier()

    pltpu.sync_copy(packed_ref.at[s], pb0)
    for k in range(GCH):
      pltpu.async_copy(src_tbl.at[pb0.at[0, k]],
                       gb0.at[pl.ds(k * CHUNK, CHUNK)], gs0.at[k])
    pltpu.async_copy(packed_ref.at[NS + s], pb1, isem.at[1])

    def phase(m, t):
      j = 6 * m + t
      g = t % 3
      g1 = (t + 1) % 3
      gx, gsx, ssx, pj = gbufs[g], gsems[g], ssems[g], pbufs[t]
      gy, gsy, ssy = gbufs[g1], gsems[g1], ssems[g1]
      pj1 = pbufs[(t + 1) % 6]
      pj2 = pbufs[(t + 2) % 6]

      def fire_next():
        pltpu.make_async_copy(packed_ref.at[(j + 1) * NS + s], pj1,
                              isem.at[(t + 1) % 6]).wait()
        for k in range(GCH):
          @pl.when(j > 1)
          def _(k=k):
            pltpu.make_async_copy(gy.at[pl.ds(k * CHUNK, CHUNK)],
                                  acc.at[pj1.at[1, k]], ssy.at[k]).wait()
          pltpu.async_copy(src_tbl.at[pj1.at[0, k]],
                           gy.at[pl.ds(k * CHUNK, CHUNK)], gsy.at[k])

      if t == 5:
        @pl.when(m < M_LOOPS - 1)
        def _():
          fire_next()
      else:
        fire_next()

      def prefetch():
        pltpu.async_copy(packed_ref.at[(j + 2) * NS + s], pj2,
                         isem.at[(t + 2) % 6])

      if t >= 4:
        @pl.when(m < M_LOOPS - 1)
        def _():
          prefetch()
      else:
        prefetch()

      k2 = jnp.full((16,), 2, jnp.int32)
      for k in range(GCH):
        pltpu.make_async_copy(src_tbl.at[pj.at[0, k]],
                              gx.at[pl.ds(k * CHUNK, CHUNK)],
                              gsx.at[k]).wait()
        kb = k * CHUNK
        kk = jnp.full((16,), k, jnp.int32)

        @plsc.parallel_loop(0, CHUNK, unroll=8)
        def _(e, kb=kb, k2=k2, kk=kk):
          v = plsc.bitcast(
              plsc.load_gather(pj, [k2, kk, jnp.full((16,), e, jnp.int32)]),
              jnp.float32)
          gx[kb + e, pl.ds(0, 16)] = gx[kb + e, pl.ds(0, 16)] * v
          gx[kb + e, pl.ds(16, 16)] = gx[kb + e, pl.ds(16, 16)] * v

        pltpu.async_copy(gx.at[pl.ds(kb, CHUNK)], acc.at[pj.at[1, k]],
                         ssx.at[k], add=True)

    @pl.loop(0, M_LOOPS)
    def _(m):
      for t in range(6):
        phase(m, t)

    for k in range(GCH):
      for t in range(3):
        pltpu.make_async_copy(gbufs[t].at[pl.ds(k * CHUNK, CHUNK)],
                              acc.at[pbufs[3 + t].at[1, k]],
                              ssems[t].at[k]).wait()

    plsc.subcore_barrier()

    @pl.loop(s, N_PIECES, step=NS)
    def _(j):
      pltpu.sync_copy(acc.at[pl.ds(j * PIECE, PIECE)],
                      dst_tbl.at[pl.ds(j * PIECE, PIECE)])

  t0 = init_ref.at[c]
  t1 = l1_ref.at[c]
  t2 = l2_ref.at[c]
  t3 = l3_ref.at[c]
  propagate(t0, t1)
  plsc.subcore_barrier()
  propagate(t1, t2)
  plsc.subcore_barrier()
  propagate(t2, t3)
  plsc.subcore_barrier()

  tables = (t0, t1, t2, t3)
  for p in range(P_LOOPS):
    base = s * P_SUB + p * PCHUNK
    pltpu.sync_copy(users_ref.at[pl.ds(base, PCHUNK)], uv)
    pltpu.sync_copy(items_ref.at[pl.ds(base, PCHUNK)], iv)

    @pl.loop(0, PCHUNK, step=16)
    def _(t):
      iv[pl.ds(t, 16)] = iv[pl.ds(t, 16)] + N_USERS

    descs = []
    for t in range(4):
      descs.append(pltpu.async_copy(
          tables[t].at[uv], gb0.at[pl.ds(t * PCHUNK, PCHUNK)],
          gsems[t % 3].at[t // 3]))
      descs.append(pltpu.async_copy(
          tables[t].at[iv], gb1.at[pl.ds(t * PCHUNK, PCHUNK)],
          ssems[t % 3].at[t // 3]))
    for d_ in descs:
      d_.wait()

    @pl.loop(0, PCHUNK)
    def _(e, p=p):
      ulo = (gb0[0 * PCHUNK + e, pl.ds(0, 16)] +
             gb0[1 * PCHUNK + e, pl.ds(0, 16)] +
             gb0[2 * PCHUNK + e, pl.ds(0, 16)] +
             gb0[3 * PCHUNK + e, pl.ds(0, 16)])
      uhi = (gb0[0 * PCHUNK + e, pl.ds(16, 16)] +
             gb0[1 * PCHUNK + e, pl.ds(16, 16)] +
             gb0[2 * PCHUNK + e, pl.ds(16, 16)] +
             gb0[3 * PCHUNK + e, pl.ds(16, 16)])
      ilo = (gb1[0 * PCHUNK + e, pl.ds(0, 16)] +
             gb1[1 * PCHUNK + e, pl.ds(0, 16)] +
             gb1[2 * PCHUNK + e, pl.ds(0, 16)] +
             gb1[3 * PCHUNK + e, pl.ds(0, 16)])
      ihi = (gb1[0 * PCHUNK + e, pl.ds(16, 16)] +
             gb1[1 * PCHUNK + e, pl.ds(16, 16)] +
             gb1[2 * PCHUNK + e, pl.ds(16, 16)] +
             gb1[3 * PCHUNK + e, pl.ds(16, 16)])
      prod = ulo * ilo + uhi * ihi
      cs = plsc.cumsum(prod)
      lane = lax.broadcasted_iota(jnp.int32, (16,), 0)
      plsc.store_scatter(gammav,
                         [jnp.full((16,), p * PCHUNK + e, jnp.int32)],
                         cs, mask=lane == 15)

  pltpu.sync_copy(gammav, gamma_ref.at[c, pl.ds(s * P_SUB, P_SUB)])


_SCRATCH = [
    pltpu.VMEM_SHARED((N_NODES, HALF), jnp.float32),
    pltpu.VMEM((3, GCH, CHUNK), jnp.int32),
    pltpu.VMEM((3, GCH, CHUNK), jnp.int32),
    pltpu.VMEM((3, GCH, CHUNK), jnp.int32),
    pltpu.VMEM((3, GCH, CHUNK), jnp.int32),
    pltpu.VMEM((3, GCH, CHUNK), jnp.int32),
    pltpu.VMEM((3, GCH, CHUNK), jnp.int32),
    pltpu.VMEM((E_SLAB, HALF), jnp.float32),
    pltpu.VMEM((E_SLAB, HALF), jnp.float32),
    pltpu.VMEM((E_SLAB, HALF), jnp.float32),
    pltpu.VMEM((PCHUNK,), jnp.int32),
    pltpu.VMEM((PCHUNK,), jnp.int32),
    pltpu.VMEM((P_SUB,), jnp.float32),
    pltpu.SemaphoreType.DMA((GCH,)),
    pltpu.SemaphoreType.DMA((GCH,)),
    pltpu.SemaphoreType.DMA((GCH,)),
    pltpu.SemaphoreType.DMA((GCH,)),
    pltpu.SemaphoreType.DMA((GCH,)),
    pltpu.SemaphoreType.DMA((GCH,)),
    pltpu.SemaphoreType.DMA((6,)),
]

_OUT = (
    jax.ShapeDtypeStruct((NC, BATCH), jnp.float32),
    jax.ShapeDtypeStruct((NC, N_NODES, HALF), jnp.float32),
    jax.ShapeDtypeStruct((NC, N_NODES, HALF), jnp.float32),
    jax.ShapeDtypeStruct((NC, N_NODES, HALF), jnp.float32),
)


def _combine_body(p_ref, o_ref):
  o_ref[...] = (p_ref[0] + p_ref[1]) * jnp.float32(1.0 / 16.0)


def kernel(users, items, user_emb_weight, item_emb_weight, edge_index,
           graph_values):
  all_emb = jnp.concatenate([user_emb_weight, item_emb_weight], axis=0)
  init = jnp.stack([all_emb[:, :HALF], all_emb[:, HALF:]])
  pad = N_EDGES_PAD - N_EDGES
  cols = jnp.concatenate(
      [edge_index[1], jnp.zeros((pad,), jnp.int32)]).reshape(
          N_SLABS, GCH, CHUNK)
  rows = jnp.concatenate(
      [edge_index[0], jnp.zeros((pad,), jnp.int32)]).reshape(
          N_SLABS, GCH, CHUNK)
  vals = lax.bitcast_convert_type(
      jnp.concatenate([graph_values, jnp.zeros((pad,), jnp.float32)]),
      jnp.int32).reshape(N_SLABS, GCH, CHUNK)
  packed = jnp.stack([cols, rows, vals], axis=1)

  mesh = plsc.VectorSubcoreMesh(core_axis_name="c", subcore_axis_name="s",
                                num_cores=NC, num_subcores=NS)
  sc = pl.kernel(_sc_body, out_type=_OUT, mesh=mesh, scratch_types=_SCRATCH,
                 compiler_params=pltpu.CompilerParams(
                     needs_layout_passes=False,
                     use_tc_tiling_on_sc=False))
  gamma_p, _, _, _ = sc(init, packed, users, items)

  out = pl.pallas_call(
      _combine_body,
      out_shape=jax.ShapeDtypeStruct((128, 128), jnp.float32))(
          gamma_p.reshape(NC, 128, 128))
  return out.reshape(BATCH)


# speedup vs baseline: 1.5529x; 1.5529x over previous
import jax
import jax.numpy as jnp
from jax import lax
from jax.experimental import pallas as pl
from jax.experimental.pallas import tpu as pltpu
from jax.experimental.pallas import tpu_sc as plsc

N_USERS = 25000
N_ITEMS = 25000
N_NODES = N_USERS + N_ITEMS
N_EDGES = 800000
HALF = 32
BATCH = 16384

NC = 2
NS = 16
CHUNK = 128
SLAB = 6
E_SLAB = CHUNK * SLAB
N_EDGES_PAD = 811008
N_SLABS = N_EDGES_PAD // E_SLAB
SLABS_SUB = N_SLABS // NS
PIECE = 400
N_PIECES = N_NODES // PIECE
P_SUB = BATCH // NS
PCHUNK = 64
P_LOOPS = P_SUB // PCHUNK


def _sc_body(init_ref, packed_ref, users_ref, items_ref,
             gamma_ref, l1_ref, l2_ref, l3_ref,
             acc, pbufa, pbufb, gath, uv, iv, gammav, gsem, ssem, isem):
  c = lax.axis_index("c")
  s = lax.axis_index("s")

  def propagate(src_tbl, dst_tbl):
    @pl.loop(0, PIECE)
    def _(r):
      gath[r, pl.ds(0, 16)] = jnp.zeros((16,), jnp.float32)
      gath[r, pl.ds(16, 16)] = jnp.zeros((16,), jnp.float32)

    @pl.loop(s, N_PIECES, step=NS)
    def _(j):
      pltpu.sync_copy(gath.at[pl.ds(0, PIECE)],
                      acc.at[pl.ds(j * PIECE, PIECE)])
    plsc.subcore_barrier()

    pltpu.sync_copy(packed_ref.at[s], pbufa)

    def do_slab(cur, nxt, jj):
      for k in range(SLAB):
        @pl.when(jj > 0)
        def _(k=k):
          pltpu.make_async_copy(
              gath.at[pl.ds(k * CHUNK, CHUNK)],
              acc.at[cur.at[1, k]], ssem.at[k]).wait()
        pltpu.async_copy(src_tbl.at[cur.at[0, k]],
                         gath.at[pl.ds(k * CHUNK, CHUNK)], gsem.at[k])

      @pl.when(jj + 1 < SLABS_SUB)
      def _():
        pltpu.async_copy(packed_ref.at[(jj + 1) * NS + s], nxt, isem)

      k2 = jnp.full((16,), 2, jnp.int32)
      for k in range(SLAB):
        pltpu.make_async_copy(src_tbl.at[cur.at[0, k]],
                              gath.at[pl.ds(k * CHUNK, CHUNK)],
                              gsem.at[k]).wait()
        kb = k * CHUNK
        kk = jnp.full((16,), k, jnp.int32)

        @plsc.parallel_loop(0, CHUNK, unroll=8)
        def _(e, kb=kb, k2=k2, kk=kk):
          v = plsc.bitcast(
              plsc.load_gather(cur, [k2, kk, jnp.full((16,), e, jnp.int32)]),
              jnp.float32)
          gath[kb + e, pl.ds(0, 16)] = gath[kb + e, pl.ds(0, 16)] * v
          gath[kb + e, pl.ds(16, 16)] = gath[kb + e, pl.ds(16, 16)] * v

        pltpu.async_copy(gath.at[pl.ds(kb, CHUNK)], acc.at[cur.at[1, k]],
                         ssem.at[k], add=True)

      @pl.when(jj + 1 < SLABS_SUB)
      def _():
        pltpu.make_async_copy(packed_ref.at[(jj + 1) * NS + s], nxt,
                              isem).wait()

    @pl.loop(0, SLABS_SUB // 2)
    def _(m):
      do_slab(pbufa, pbufb, 2 * m)
      do_slab(pbufb, pbufa, 2 * m + 1)

    for k in range(SLAB):
      pltpu.make_async_copy(gath.at[pl.ds(k * CHUNK, CHUNK)],
                            acc.at[pbufb.at[1, k]], ssem.at[k]).wait()

    plsc.subcore_barrier()

    @pl.loop(s, N_PIECES, step=NS)
    def _(j):
      pltpu.sync_copy(acc.at[pl.ds(j * PIECE, PIECE)],
                      dst_tbl.at[pl.ds(j * PIECE, PIECE)])

  t0 = init_ref.at[c]
  t1 = l1_ref.at[c]
  t2 = l2_ref.at[c]
  t3 = l3_ref.at[c]
  propagate(t0, t1)
  plsc.subcore_barrier()
  propagate(t1, t2)
  plsc.subcore_barrier()
  propagate(t2, t3)
  plsc.subcore_barrier()

  tables = (t0, t1, t2, t3)
  for p in range(P_LOOPS):
    base = s * P_SUB + p * PCHUNK
    pltpu.sync_copy(users_ref.at[pl.ds(base, PCHUNK)], uv)
    pltpu.sync_copy(items_ref.at[pl.ds(base, PCHUNK)], iv)

    @pl.loop(0, PCHUNK, step=16)
    def _(t):
      iv[pl.ds(t, 16)] = iv[pl.ds(t, 16)] + N_USERS

    descs = []
    for t in range(4):
      descs.append(pltpu.async_copy(
          tables[t].at[uv], gath.at[pl.ds(t * PCHUNK, PCHUNK)],
          gsem.at[t % SLAB]))
      descs.append(pltpu.async_copy(
          tables[t].at[iv], gath.at[pl.ds(4 * PCHUNK + t * PCHUNK, PCHUNK)],
          ssem.at[t % SLAB]))
    for d_ in descs:
      d_.wait()

    @pl.loop(0, PCHUNK)
    def _(e, p=p):
      ulo = (gath[0 * PCHUNK + e, pl.ds(0, 16)] +
             gath[1 * PCHUNK + e, pl.ds(0, 16)] +
             gath[2 * PCHUNK + e, pl.ds(0, 16)] +
             gath[3 * PCHUNK + e, pl.ds(0, 16)])
      uhi = (gath[0 * PCHUNK + e, pl.ds(16, 16)] +
             gath[1 * PCHUNK + e, pl.ds(16, 16)] +
             gath[2 * PCHUNK + e, pl.ds(16, 16)] +
             gath[3 * PCHUNK + e, pl.ds(16, 16)])
      ilo = (gath[4 * PCHUNK + e, pl.ds(0, 16)] +
             gath[5 * PCHUNK + e, pl.ds(0, 16)] +
             gath[6 * PCHUNK + e, pl.ds(0, 16)] +
             gath[7 * PCHUNK + e, pl.ds(0, 16)])
      ihi = (gath[4 * PCHUNK + e, pl.ds(16, 16)] +
             gath[5 * PCHUNK + e, pl.ds(16, 16)] +
             gath[6 * PCHUNK + e, pl.ds(16, 16)] +
             gath[7 * PCHUNK + e, pl.ds(16, 16)])
      prod = ulo * ilo + uhi * ihi
      cs = plsc.cumsum(prod)
      lane = lax.broadcasted_iota(jnp.int32, (16,), 0)
      plsc.store_scatter(gammav,
                         [jnp.full((16,), p * PCHUNK + e, jnp.int32)],
                         cs, mask=lane == 15)

  pltpu.sync_copy(gammav, gamma_ref.at[c, pl.ds(s * P_SUB, P_SUB)])


_SCRATCH = [
    pltpu.VMEM_SHARED((N_NODES, HALF), jnp.float32),
    pltpu.VMEM((3, SLAB, CHUNK), jnp.int32),
    pltpu.VMEM((3, SLAB, CHUNK), jnp.int32),
    pltpu.VMEM((E_SLAB, HALF), jnp.float32),
    pltpu.VMEM((PCHUNK,), jnp.int32),
    pltpu.VMEM((PCHUNK,), jnp.int32),
    pltpu.VMEM((P_SUB,), jnp.float32),
    pltpu.SemaphoreType.DMA((SLAB,)),
    pltpu.SemaphoreType.DMA((SLAB,)),
    pltpu.SemaphoreType.DMA,
]

_OUT = (
    jax.ShapeDtypeStruct((NC, BATCH), jnp.float32),
    jax.ShapeDtypeStruct((NC, N_NODES, HALF), jnp.float32),
    jax.ShapeDtypeStruct((NC, N_NODES, HALF), jnp.float32),
    jax.ShapeDtypeStruct((NC, N_NODES, HALF), jnp.float32),
)


def _combine_body(p_ref, o_ref):
  o_ref[...] = (p_ref[0] + p_ref[1]) * jnp.float32(1.0 / 16.0)


def kernel(users, items, user_emb_weight, item_emb_weight, edge_index,
           graph_values):
  all_emb = jnp.concatenate([user_emb_weight, item_emb_weight], axis=0)
  init = jnp.stack([all_emb[:, :HALF], all_emb[:, HALF:]])
  pad = N_EDGES_PAD - N_EDGES
  cols = jnp.concatenate(
      [edge_index[1], jnp.zeros((pad,), jnp.int32)]).reshape(
          N_SLABS, SLAB, CHUNK)
  rows = jnp.concatenate(
      [edge_index[0], jnp.zeros((pad,), jnp.int32)]).reshape(
          N_SLABS, SLAB, CHUNK)
  vals = lax.bitcast_convert_type(
      jnp.concatenate([graph_values, jnp.zeros((pad,), jnp.float32)]),
      jnp.int32).reshape(N_SLABS, SLAB, CHUNK)
  packed = jnp.stack([cols, rows, vals], axis=1)

  mesh = plsc.VectorSubcoreMesh(core_axis_name="c", subcore_axis_name="s",
                                num_cores=NC, num_subcores=NS)
  sc = pl.kernel(_sc_body, out_type=_OUT, mesh=mesh, scratch_types=_SCRATCH,
                 compiler_params=pltpu.CompilerParams(
                     needs_layout_passes=False,
                     use_tc_tiling_on_sc=False))
  gamma_p, _, _, _ = sc(init, packed, users, items)

  out = pl.pallas_call(
      _combine_body,
      out_shape=jax.ShapeDtypeStruct((128, 128), jnp.float32))(
          gamma_p.reshape(NC, 128, 128))
  return out.reshape(BATCH)


# speedup vs baseline: 1.5622x; 1.0060x over previous
import jax
import jax.numpy as jnp
from jax import lax
from jax.experimental import pallas as pl
from jax.experimental.pallas import tpu as pltpu
from jax.experimental.pallas import tpu_sc as plsc

N_USERS = 25000
N_ITEMS = 25000
N_NODES = N_USERS + N_ITEMS
N_EDGES = 800000
HALF = 32
BATCH = 16384

NC = 2
NS = 16
CHUNK = 128
SLAB = 6
E_SLAB = CHUNK * SLAB
N_EDGES_PAD = 811008
N_SLABS = N_EDGES_PAD // E_SLAB
SLABS_SUB = N_SLABS // NS
PIECE = 400
N_PIECES = N_NODES // PIECE
P_SUB = BATCH // NS
PCHUNK = 64
P_LOOPS = P_SUB // PCHUNK


def _sc_body(init_ref, packed_ref, users_ref, items_ref,
             gamma_ref, l1_ref, l2_ref, l3_ref,
             acc, pbufa, pbufb, gath, uv, iv, gammav, gsem, ssem, isem):
  c = lax.axis_index("c")
  s = lax.axis_index("s")

  @pl.loop(0, PIECE)
  def _(r):
    gath[r, pl.ds(0, 16)] = jnp.zeros((16,), jnp.float32)
    gath[r, pl.ds(16, 16)] = jnp.zeros((16,), jnp.float32)

  @pl.loop(s, N_PIECES, step=NS)
  def _(j):
    pltpu.sync_copy(gath.at[pl.ds(0, PIECE)],
                    acc.at[pl.ds(j * PIECE, PIECE)])
  plsc.subcore_barrier()

  def propagate(src_tbl, dst_tbl, rezero):
    pltpu.sync_copy(packed_ref.at[s], pbufa)

    def do_slab(cur, nxt, jj):
      for k in range(SLAB):
        @pl.when(jj > 0)
        def _(k=k):
          pltpu.make_async_copy(
              gath.at[pl.ds(k * CHUNK, CHUNK)],
              acc.at[cur.at[1, k]], ssem.at[k]).wait()
        pltpu.async_copy(src_tbl.at[cur.at[0, k]],
                         gath.at[pl.ds(k * CHUNK, CHUNK)], gsem.at[k])

      @pl.when(jj + 1 < SLABS_SUB)
      def _():
        pltpu.async_copy(packed_ref.at[(jj + 1) * NS + s], nxt, isem)

      k2 = jnp.full((16,), 2, jnp.int32)
      for k in range(SLAB):
        pltpu.make_async_copy(src_tbl.at[cur.at[0, k]],
                              gath.at[pl.ds(k * CHUNK, CHUNK)],
                              gsem.at[k]).wait()
        kb = k * CHUNK
        kk = jnp.full((16,), k, jnp.int32)

        @plsc.parallel_loop(0, CHUNK, unroll=8)
        def _(e, kb=kb, k2=k2, kk=kk):
          v = plsc.bitcast(
              plsc.load_gather(cur, [k2, kk, jnp.full((16,), e, jnp.int32)]),
              jnp.float32)
          gath[kb + e, pl.ds(0, 16)] = gath[kb + e, pl.ds(0, 16)] * v
          gath[kb + e, pl.ds(16, 16)] = gath[kb + e, pl.ds(16, 16)] * v

        pltpu.async_copy(gath.at[pl.ds(kb, CHUNK)], acc.at[cur.at[1, k]],
                         ssem.at[k], add=True)

      @pl.when(jj + 1 < SLABS_SUB)
      def _():
        pltpu.make_async_copy(packed_ref.at[(jj + 1) * NS + s], nxt,
                              isem).wait()

    @pl.loop(0, SLABS_SUB // 2)
    def _(m):
      do_slab(pbufa, pbufb, 2 * m)
      do_slab(pbufb, pbufa, 2 * m + 1)

    for k in range(SLAB):
      pltpu.make_async_copy(gath.at[pl.ds(k * CHUNK, CHUNK)],
                            acc.at[pbufb.at[1, k]], ssem.at[k]).wait()

    if rezero:
      @pl.loop(0, PIECE)
      def _(r):
        gath[r, pl.ds(0, 16)] = jnp.zeros((16,), jnp.float32)
        gath[r, pl.ds(16, 16)] = jnp.zeros((16,), jnp.float32)

    plsc.subcore_barrier()

    @pl.loop(s, N_PIECES, step=NS)
    def _(j):
      pltpu.sync_copy(acc.at[pl.ds(j * PIECE, PIECE)],
                      dst_tbl.at[pl.ds(j * PIECE, PIECE)])
      if rezero:
        pltpu.async_copy(gath.at[pl.ds(0, PIECE)],
                         acc.at[pl.ds(j * PIECE, PIECE)], isem)

    if rezero:
      @pl.loop(s, N_PIECES, step=NS)
      def _(j):
        pltpu.make_async_copy(gath.at[pl.ds(0, PIECE)],
                              acc.at[pl.ds(j * PIECE, PIECE)], isem).wait()

  t0 = init_ref.at[c]
  t1 = l1_ref.at[c]
  t2 = l2_ref.at[c]
  t3 = l3_ref.at[c]
  propagate(t0, t1, rezero=True)
  plsc.subcore_barrier()
  propagate(t1, t2, rezero=True)
  plsc.subcore_barrier()
  propagate(t2, t3, rezero=False)
  plsc.subcore_barrier()

  tables = (t0, t1, t2, t3)
  for p in range(P_LOOPS):
    base = s * P_SUB + p * PCHUNK
    pltpu.sync_copy(users_ref.at[pl.ds(base, PCHUNK)], uv)
    pltpu.sync_copy(items_ref.at[pl.ds(base, PCHUNK)], iv)

    @pl.loop(0, PCHUNK, step=16)
    def _(t):
      iv[pl.ds(t, 16)] = iv[pl.ds(t, 16)] + N_USERS

    descs = []
    for t in range(4):
      descs.append(pltpu.async_copy(
          tables[t].at[uv], gath.at[pl.ds(t * PCHUNK, PCHUNK)],
          gsem.at[t % SLAB]))
      descs.append(pltpu.async_copy(
          tables[t].at[iv], gath.at[pl.ds(4 * PCHUNK + t * PCHUNK, PCHUNK)],
          ssem.at[t % SLAB]))
    for d_ in descs:
      d_.wait()

    @pl.loop(0, PCHUNK)
    def _(e, p=p):
      ulo = (gath[0 * PCHUNK + e, pl.ds(0, 16)] +
             gath[1 * PCHUNK + e, pl.ds(0, 16)] +
             gath[2 * PCHUNK + e, pl.ds(0, 16)] +
             gath[3 * PCHUNK + e, pl.ds(0, 16)])
      uhi = (gath[0 * PCHUNK + e, pl.ds(16, 16)] +
             gath[1 * PCHUNK + e, pl.ds(16, 16)] +
             gath[2 * PCHUNK + e, pl.ds(16, 16)] +
             gath[3 * PCHUNK + e, pl.ds(16, 16)])
      ilo = (gath[4 * PCHUNK + e, pl.ds(0, 16)] +
             gath[5 * PCHUNK + e, pl.ds(0, 16)] +
             gath[6 * PCHUNK + e, pl.ds(0, 16)] +
             gath[7 * PCHUNK + e, pl.ds(0, 16)])
      ihi = (gath[4 * PCHUNK + e, pl.ds(16, 16)] +
             gath[5 * PCHUNK + e, pl.ds(16, 16)] +
             gath[6 * PCHUNK + e, pl.ds(16, 16)] +
             gath[7 * PCHUNK + e, pl.ds(16, 16)])
      prod = ulo * ilo + uhi * ihi
      cs = plsc.cumsum(prod)
      lane = lax.broadcasted_iota(jnp.int32, (16,), 0)
      plsc.store_scatter(gammav,
                         [jnp.full((16,), p * PCHUNK + e, jnp.int32)],
                         cs, mask=lane == 15)

  pltpu.sync_copy(gammav, gamma_ref.at[c, pl.ds(s * P_SUB, P_SUB)])


_SCRATCH = [
    pltpu.VMEM_SHARED((N_NODES, HALF), jnp.float32),
    pltpu.VMEM((3, SLAB, CHUNK), jnp.int32),
    pltpu.VMEM((3, SLAB, CHUNK), jnp.int32),
    pltpu.VMEM((E_SLAB, HALF), jnp.float32),
    pltpu.VMEM((PCHUNK,), jnp.int32),
    pltpu.VMEM((PCHUNK,), jnp.int32),
    pltpu.VMEM((P_SUB,), jnp.float32),
    pltpu.SemaphoreType.DMA((SLAB,)),
    pltpu.SemaphoreType.DMA((SLAB,)),
    pltpu.SemaphoreType.DMA,
]

_OUT = (
    jax.ShapeDtypeStruct((NC, BATCH), jnp.float32),
    jax.ShapeDtypeStruct((NC, N_NODES, HALF), jnp.float32),
    jax.ShapeDtypeStruct((NC, N_NODES, HALF), jnp.float32),
    jax.ShapeDtypeStruct((NC, N_NODES, HALF), jnp.float32),
)


def _combine_body(p_ref, o_ref):
  o_ref[...] = (p_ref[0] + p_ref[1]) * jnp.float32(1.0 / 16.0)


def kernel(users, items, user_emb_weight, item_emb_weight, edge_index,
           graph_values):
  all_emb = jnp.concatenate([user_emb_weight, item_emb_weight], axis=0)
  init = jnp.stack([all_emb[:, :HALF], all_emb[:, HALF:]])
  pad = N_EDGES_PAD - N_EDGES
  cols = jnp.concatenate(
      [edge_index[1], jnp.zeros((pad,), jnp.int32)]).reshape(
          N_SLABS, SLAB, CHUNK)
  rows = jnp.concatenate(
      [edge_index[0], jnp.zeros((pad,), jnp.int32)]).reshape(
          N_SLABS, SLAB, CHUNK)
  vals = lax.bitcast_convert_type(
      jnp.concatenate([graph_values, jnp.zeros((pad,), jnp.float32)]),
      jnp.int32).reshape(N_SLABS, SLAB, CHUNK)
  packed = jnp.stack([cols, rows, vals], axis=1)

  mesh = plsc.VectorSubcoreMesh(core_axis_name="c", subcore_axis_name="s",
                                num_cores=NC, num_subcores=NS)
  sc = pl.kernel(_sc_body, out_type=_OUT, mesh=mesh, scratch_types=_SCRATCH,
                 compiler_params=pltpu.CompilerParams(
                     needs_layout_passes=False,
                     use_tc_tiling_on_sc=False))
  gamma_p, _, _, _ = sc(init, packed, users, items)

  out = pl.pallas_call(
      _combine_body,
      out_shape=jax.ShapeDtypeStruct((128, 128), jnp.float32))(
          gamma_p.reshape(NC, 128, 128))
  return out.reshape(BATCH)
